# Initial kernel scaffold; baseline (speedup 1.0000x reference)
import jax
import jax.numpy as jnp
from jax.experimental import pallas as pl


def kernel(x, seg, word_table, pos_table, type_table, gamma, beta):
    raise NotImplementedError("write your pallas kernel here")


# speedup vs baseline: 3.5617x; 3.5617x over previous
import functools

import jax
import jax.numpy as jnp
from jax import lax
from jax.experimental import pallas as pl
from jax.experimental.pallas import tpu as pltpu
from jax.experimental.pallas import tpu_sc as plsc

B = 1024
L = 200
D = 128
EPS = 1e-12
LANES = 16
NJ = D // LANES
NC = 2
NS = 16
NW = NC * NS
ROWS_PER_TILE = B // NW
GSPLIT = 128


def _allreduce_sum(v):
    dnums = lax.GatherDimensionNumbers(
        offset_dims=(), collapsed_slice_dims=(0,), start_index_map=(0,))
    for step in (1, 2, 4, 8):
        idx = (jnp.arange(16, dtype=jnp.int32) ^ step)[:, None]
        v = v + lax.gather(v, idx, dnums, slice_sizes=(1,),
                           mode=lax.GatherScatterMode.PROMISE_IN_BOUNDS)
    return v


def _rsqrt_nr(x):
    i = lax.bitcast_convert_type(x, jnp.int32)
    i = jnp.int32(0x5F3759DF) - lax.shift_right_logical(i, 1)
    y = lax.bitcast_convert_type(i, jnp.float32)
    for _ in range(3):
        y = y * (1.5 - 0.5 * x * y * y)
    return y


def _make_kernel():
    mesh = plsc.VectorSubcoreMesh(core_axis_name="c", subcore_axis_name="s")

    @functools.partial(
        pl.kernel,
        mesh=mesh,
        out_type=jax.ShapeDtypeStruct((B * L, D), jnp.float32),
        scratch_types=[
            pltpu.VMEM((2, L, D), jnp.float32),
            pltpu.VMEM((2, L, D), jnp.float32),
            pltpu.VMEM((L,), jnp.int32),
            pltpu.VMEM((L,), jnp.int32),
            pltpu.VMEM((L,), jnp.int32),
            pltpu.VMEM((L,), jnp.int32),
            pltpu.VMEM((D,), jnp.float32),
            pltpu.VMEM((D,), jnp.float32),
            pltpu.VMEM((2, D), jnp.float32),
            pltpu.SemaphoreType.DMA,
            pltpu.SemaphoreType.DMA,
            pltpu.SemaphoreType.DMA,
            pltpu.SemaphoreType.DMA,
        ],
    )
    def emb_kernel(x_hbm, seg_hbm, word_hbm, pos_hbm, type_hbm, gamma_hbm,
                   beta_hbm, out_hbm, rows_v, base_v, idx_v0, idx_v1, seg_v0,
                   seg_v1, gamma_v, beta_v, type_v, sem_g0, sem_g1, sem_o0,
                   sem_o1):
        sem_g = (sem_g0, sem_g1)
        sem_o = (sem_o0, sem_o1)
        idx_v = (idx_v0, idx_v1)
        seg_v = (seg_v0, seg_v1)
        wid = lax.axis_index("s") * NC + lax.axis_index("c")
        row0 = wid * ROWS_PER_TILE

        pltpu.sync_copy(gamma_hbm, gamma_v)
        pltpu.sync_copy(beta_hbm, beta_v)
        pltpu.sync_copy(type_hbm, type_v)
        pltpu.sync_copy(pos_hbm.at[pl.ds(0, L)], base_v.at[0])
        pltpu.sync_copy(pos_hbm.at[pl.ds(0, L)], base_v.at[1])

        def init_body(l, c):
            for j in range(NJ):
                sl = pl.ds(j * LANES, LANES)
                base_v[0, l, sl] = base_v[0, l, sl] + type_v[0, sl]
                base_v[1, l, sl] = base_v[1, l, sl] + type_v[1, sl]
            return c
        lax.fori_loop(0, L, init_body, 0)

        def start_gather(g, d):
            tok0 = (row0 + g) * L
            pltpu.sync_copy(x_hbm.at[pl.ds(tok0, L)], idx_v[d])
            pltpu.sync_copy(seg_hbm.at[pl.ds(tok0, L)], seg_v[d])
            pltpu.async_copy(word_hbm.at[idx_v[d].at[pl.ds(0, GSPLIT)]],
                             rows_v.at[d, pl.ds(0, GSPLIT)], sem_g[d])
            pltpu.async_copy(word_hbm.at[idx_v[d].at[pl.ds(GSPLIT, L - GSPLIT)]],
                             rows_v.at[d, pl.ds(GSPLIT, L - GSPLIT)], sem_g[d])

        def wait_gather(d):
            pltpu.make_async_copy(word_hbm.at[idx_v[d].at[pl.ds(0, GSPLIT)]],
                                  rows_v.at[d, pl.ds(0, GSPLIT)],
                                  sem_g[d]).wait()
            pltpu.make_async_copy(word_hbm.at[idx_v[d].at[pl.ds(GSPLIT, L - GSPLIT)]],
                                  rows_v.at[d, pl.ds(GSPLIT, L - GSPLIT)],
                                  sem_g[d]).wait()

        def start_out(g, d):
            tok0 = (row0 + g) * L
            pltpu.async_copy(rows_v.at[d], out_hbm.at[pl.ds(tok0, L)], sem_o[d])

        def wait_out(d):
            pltpu.make_async_copy(rows_v.at[d], out_hbm.at[pl.ds(0, L)],
                                  sem_o[d]).wait()

        def compute_row(d):
            hoisted = tuple(gamma_v[pl.ds(j * LANES, LANES)] for j in range(NJ)) \
                + tuple(beta_v[pl.ds(j * LANES, LANES)] for j in range(NJ))

            def one_token(t, s, carry):
                acc1 = jnp.zeros((LANES,), jnp.float32)
                acc2 = jnp.zeros((LANES,), jnp.float32)
                e = []
                for j in range(NJ):
                    sl = pl.ds(j * LANES, LANES)
                    ej = rows_v[d, t, sl] + base_v[s, t, sl]
                    e.append(ej)
                    acc1 = acc1 + ej
                    acc2 = acc2 + ej * ej
                meanv = _allreduce_sum(acc1) * (1.0 / D)
                varv = _allreduce_sum(acc2) * (1.0 / D) - meanv * meanv + EPS
                inv = _rsqrt_nr(varv)
                for j in range(NJ):
                    sl = pl.ds(j * LANES, LANES)
                    nj = (e[j] - meanv) * inv
                    rows_v[d, t, sl] = nj * carry[j] + carry[NJ + j]

            def grp_body(g, carry):
                t0 = g * LANES
                segs = seg_v[d][pl.ds(t0, LANES)]
                for k in range(LANES):
                    one_token(t0 + k, segs[k], carry)
                return carry

            carry = lax.fori_loop(0, L // LANES, grp_body, hoisted)
            tail = L - (L // LANES) * LANES
            if tail:
                t0 = L - LANES
                segs = seg_v[d][pl.ds(t0, LANES)]
                for k in range(LANES - tail, LANES):
                    one_token(t0 + k, segs[k], carry)

        start_gather(0, 0)

        def main_body(i, c):
            for dd in range(2):
                g = 2 * i + dd
                dn = 1 - dd

                @pl.when(g + 1 < ROWS_PER_TILE)
                def _prefetch():
                    @pl.when(g >= 1)
                    def _drain():
                        wait_out(dn)
                    start_gather(g + 1, dn)

                wait_gather(dd)
                compute_row(dd)
                start_out(g, dd)
            return c

        lax.fori_loop(0, ROWS_PER_TILE // 2, main_body, 0)
        wait_out(0)
        wait_out(1)

    return emb_kernel


_emb_kernel = _make_kernel()


def kernel(x, seg, word_table, pos_table, type_table, gamma, beta):
    xf = x.reshape(B * L).astype(jnp.int32)
    sf = seg.reshape(B * L).astype(jnp.int32)
    out = _emb_kernel(xf, sf, word_table, pos_table, type_table, gamma, beta)
    return out.reshape(B, L, D)


# speedup vs baseline: 4.0003x; 1.1232x over previous
import functools

import jax
import jax.numpy as jnp
from jax import lax
from jax.experimental import pallas as pl
from jax.experimental.pallas import tpu as pltpu
from jax.experimental.pallas import tpu_sc as plsc

B = 1024
L = 200
D = 128
EPS = 1e-12
LANES = 16
NJ = D // LANES
NC = 2
NS = 16
NW = NC * NS
ROWS_PER_TILE = B // NW
TOKS_PER_TILE = ROWS_PER_TILE * L
CA = 104
CB = L - CA
NCHUNKS = 2 * ROWS_PER_TILE


def _allreduce_sum(v):
    dnums = lax.GatherDimensionNumbers(
        offset_dims=(), collapsed_slice_dims=(0,), start_index_map=(0,))
    for step in (1, 2, 4, 8):
        idx = (jnp.arange(16, dtype=jnp.int32) ^ step)[:, None]
        v = v + lax.gather(v, idx, dnums, slice_sizes=(1,),
                           mode=lax.GatherScatterMode.PROMISE_IN_BOUNDS)
    return v


def _rsqrt_nr(x):
    i = lax.bitcast_convert_type(x, jnp.int32)
    i = jnp.int32(0x5F3759DF) - lax.shift_right_logical(i, 1)
    y = lax.bitcast_convert_type(i, jnp.float32)
    for _ in range(3):
        y = y * (1.5 - 0.5 * x * y * y)
    return y


def _make_kernel():
    mesh = plsc.VectorSubcoreMesh(core_axis_name="c", subcore_axis_name="s")

    @functools.partial(
        pl.kernel,
        mesh=mesh,
        out_type=jax.ShapeDtypeStruct((B * L, D), jnp.float32),
        scratch_types=[
            pltpu.VMEM((2, CA, D), jnp.float32),
            pltpu.VMEM((2, CA, D), jnp.float32),
            pltpu.VMEM((2, L, D), jnp.float32),
            pltpu.VMEM((TOKS_PER_TILE,), jnp.int32),
            pltpu.VMEM((TOKS_PER_TILE,), jnp.int32),
            pltpu.VMEM((D,), jnp.float32),
            pltpu.VMEM((D,), jnp.float32),
            pltpu.VMEM((2, D), jnp.float32),
            pltpu.SemaphoreType.DMA,
            pltpu.SemaphoreType.DMA,
            pltpu.SemaphoreType.DMA,
            pltpu.SemaphoreType.DMA,
        ],
    )
    def emb_kernel(x_hbm, seg_hbm, word_hbm, pos_hbm, type_hbm, gamma_hbm,
                   beta_hbm, out_hbm, grow_v, orow_v, base_v, idx_v, seg_v,
                   gamma_v, beta_v, type_v, sem_g0, sem_g1, sem_o0, sem_o1):
        sem_g = (sem_g0, sem_g1)
        sem_o = (sem_o0, sem_o1)
        wid = lax.axis_index("s") * NC + lax.axis_index("c")
        tok0 = wid * TOKS_PER_TILE

        pltpu.sync_copy(x_hbm.at[pl.ds(tok0, TOKS_PER_TILE)], idx_v)
        pltpu.sync_copy(seg_hbm.at[pl.ds(tok0, TOKS_PER_TILE)], seg_v)
        pltpu.sync_copy(gamma_hbm, gamma_v)
        pltpu.sync_copy(beta_hbm, beta_v)
        pltpu.sync_copy(type_hbm, type_v)
        pltpu.sync_copy(pos_hbm.at[pl.ds(0, L)], base_v.at[0])
        pltpu.sync_copy(pos_hbm.at[pl.ds(0, L)], base_v.at[1])

        def init_body(l, c):
            for j in range(NJ):
                sl = pl.ds(j * LANES, LANES)
                base_v[0, l, sl] = base_v[0, l, sl] + type_v[0, sl]
                base_v[1, l, sl] = base_v[1, l, sl] + type_v[1, sl]
            return c
        lax.fori_loop(0, L, init_body, 0)

        def chunk_off(c):
            return (c // 2) * L + (c % 2) * CA

        def start_gather(c, d, clen):
            off = chunk_off(c)
            pltpu.async_copy(word_hbm.at[idx_v.at[pl.ds(off, clen)]],
                             grow_v.at[d, pl.ds(0, clen)], sem_g[d])

        def wait_gather(d, clen):
            pltpu.make_async_copy(word_hbm.at[idx_v.at[pl.ds(0, clen)]],
                                  grow_v.at[d, pl.ds(0, clen)],
                                  sem_g[d]).wait()

        def start_out(c, d, clen):
            off = tok0 + chunk_off(c)
            pltpu.async_copy(orow_v.at[d, pl.ds(0, clen)],
                             out_hbm.at[pl.ds(off, clen)], sem_o[d])

        def wait_out(d, clen):
            pltpu.make_async_copy(orow_v.at[d, pl.ds(0, clen)],
                                  out_hbm.at[pl.ds(0, clen)], sem_o[d]).wait()

        def compute_chunk(c, d, clen):
            loff = (c % 2) * CA
            soff = chunk_off(c)
            hoisted = tuple(gamma_v[pl.ds(j * LANES, LANES)] for j in range(NJ)) \
                + tuple(beta_v[pl.ds(j * LANES, LANES)] for j in range(NJ))

            def one_token(tr, l, s, carry):
                acc1 = jnp.zeros((LANES,), jnp.float32)
                acc2 = jnp.zeros((LANES,), jnp.float32)
                e = []
                for j in range(NJ):
                    sl = pl.ds(j * LANES, LANES)
                    ej = grow_v[d, tr, sl] + base_v[s, l, sl]
                    e.append(ej)
                    acc1 = acc1 + ej
                    acc2 = acc2 + ej * ej
                meanv = _allreduce_sum(acc1) * (1.0 / D)
                varv = _allreduce_sum(acc2) * (1.0 / D) - meanv * meanv + EPS
                inv = _rsqrt_nr(varv)
                for j in range(NJ):
                    sl = pl.ds(j * LANES, LANES)
                    nj = (e[j] - meanv) * inv
                    orow_v[d, tr, sl] = nj * carry[j] + carry[NJ + j]

            ngrp = clen // LANES
            tail = clen - ngrp * LANES

            def _grp_body(g, carry):
                t0 = g * LANES
                segs = seg_v[pl.ds(soff + t0, LANES)]
                for k in range(LANES):
                    one_token(t0 + k, loff + t0 + k, segs[k], carry)
                return carry
            lax.fori_loop(0, ngrp, _grp_body, hoisted)

            if tail:
                t0 = clen - LANES
                segs = seg_v[pl.ds(soff + t0, LANES)]
                for k in range(LANES - tail, LANES):
                    one_token(t0 + k, loff + t0 + k, segs[k], hoisted)

        start_gather(0, 0, CA)
        start_gather(1, 1, CB)

        def main_body(i, cr):
            for dd in range(2):
                c = 2 * i + dd
                clen = CA if dd == 0 else CB
                wait_gather(dd, clen)

                @pl.when(c >= 2)
                def _drain():
                    wait_out(dd, clen)

                compute_chunk(c, dd, clen)
                start_out(c, dd, clen)

                @pl.when(c + 2 < NCHUNKS)
                def _next():
                    start_gather(c + 2, dd, clen)
            return cr

        lax.fori_loop(0, NCHUNKS // 2, main_body, 0)
        wait_out(0, CA)
        wait_out(1, CB)

    return emb_kernel


_emb_kernel = _make_kernel()


def kernel(x, seg, word_table, pos_table, type_table, gamma, beta):
    xf = x.reshape(B * L).astype(jnp.int32)
    sf = seg.reshape(B * L).astype(jnp.int32)
    out = _emb_kernel(xf, sf, word_table, pos_table, type_table, gamma, beta)
    return out.reshape(B, L, D)
